# baseline (speedup 1.0000x reference)
import functools

import jax
import jax.numpy as jnp
from jax import lax
from jax.experimental import pallas as pl
from jax.experimental.pallas import tpu as pltpu
from jax.experimental.pallas import tpu_sc as plsc

_NC = 2
_NS = 16
_NW = _NC * _NS
_LANES = 16


@functools.cache
def _build(batch, seq, d):
    b_total = batch * seq
    b_per_w = b_total // _NW

    mesh = plsc.VectorSubcoreMesh(
        core_axis_name="c", subcore_axis_name="s",
        num_cores=_NC, num_subcores=_NS,
    )

    @functools.partial(
        pl.kernel,
        mesh=mesh,
        out_type=jax.ShapeDtypeStruct((b_total, d), jnp.float32),
        scratch_types=[
            pltpu.VMEM((b_per_w,), jnp.int32),
            pltpu.VMEM((b_per_w, d), jnp.float32),
            pltpu.VMEM((b_per_w // 2, d), jnp.float32),
            [pltpu.SemaphoreType.DMA] * 8,
            pltpu.SemaphoreType.DMA,
        ],
    )
    def emb_kernel(ids_hbm, tok_hbm, pos_hbm, out_hbm, ids_v, rows_v, pos_v,
                   rsems, psem):
        wid = lax.axis_index("s") * _NC + lax.axis_index("c")
        base = wid * b_per_w
        pos_base = lax.rem(base, seq)

        pltpu.sync_copy(ids_hbm.at[wid], ids_v)
        half = b_per_w // 2
        pos_cp = pltpu.async_copy(
            pos_hbm.at[pl.ds(pos_base, half)], pos_v, psem
        )

        def fire16(i, carry):
            vec = ids_v[pl.ds(i * _LANES, _LANES)]
            for l in range(_LANES):
                tok = vec[l]
                pltpu.async_copy(
                    tok_hbm.at[pl.ds(tok, 1)],
                    rows_v.at[pl.ds(i * _LANES + l, 1)],
                    rsems[l % 8],
                )
            return carry

        lax.fori_loop(0, b_per_w // _LANES, fire16, 0)

        for q in range(8):
            pltpu.make_async_copy(
                tok_hbm.at[pl.ds(0, b_per_w // 8)],
                rows_v.at[pl.ds(q * (b_per_w // 8), b_per_w // 8)],
                rsems[q],
            ).wait()

        def half_pass(h):
            hbase = h * half
            pos_cp_h = pltpu.make_async_copy(
                pos_hbm.at[pl.ds(pos_base + hbase, half)], pos_v, psem
            )
            pos_cp_h.wait()
            if h == 0:
                pass

            def add_row(i, carry):
                for c in range(d // _LANES):
                    sl = pl.ds(c * _LANES, _LANES)
                    rows_v[hbase + i, sl] = rows_v[hbase + i, sl] + pos_v[i, sl]
                return carry

            lax.fori_loop(0, half, add_row, 0)

        half_pass(0)
        pltpu.async_copy(pos_hbm.at[pl.ds(pos_base + half, half)], pos_v, psem)
        half_pass(1)

        pltpu.sync_copy(rows_v, out_hbm.at[pl.ds(base, b_per_w)])

    return emb_kernel


def kernel(token_ids, token_table, pos_table):
    batch, seq = token_ids.shape
    d = token_table.shape[1]
    b_per_w = (batch * seq) // _NW
    ids = token_ids.astype(jnp.int32).reshape(_NW, b_per_w)
    out = _build(batch, seq, d)(ids, token_table, pos_table)
    return out.reshape(batch, seq, d)


# speedup vs baseline: 1.0016x; 1.0016x over previous
import functools

import jax
import jax.numpy as jnp
from jax import lax
from jax.experimental import pallas as pl
from jax.experimental.pallas import tpu as pltpu
from jax.experimental.pallas import tpu_sc as plsc

_NC = 2
_NS = 16
_NW = _NC * _NS
_LANES = 16


@functools.cache
def _build(batch, seq, d):
    b_total = batch * seq
    b_per_w = b_total // _NW

    mesh = plsc.VectorSubcoreMesh(
        core_axis_name="c", subcore_axis_name="s",
        num_cores=_NC, num_subcores=_NS,
    )

    @functools.partial(
        pl.kernel,
        mesh=mesh,
        compiler_params=pltpu.CompilerParams(skip_device_barrier=True),
        out_type=jax.ShapeDtypeStruct((b_total, d), jnp.float32),
        scratch_types=[
            pltpu.VMEM((b_per_w,), jnp.int32),
            pltpu.VMEM((b_per_w, d), jnp.float32),
            pltpu.VMEM((b_per_w // 2, d), jnp.float32),
            [pltpu.SemaphoreType.DMA] * 8,
            pltpu.SemaphoreType.DMA,
        ],
    )
    def emb_kernel(ids_hbm, tok_hbm, pos_hbm, out_hbm, ids_v, rows_v, pos_v,
                   rsems, psem):
        wid = lax.axis_index("s") * _NC + lax.axis_index("c")
        base = wid * b_per_w
        pos_base = lax.rem(base, seq)

        pltpu.sync_copy(ids_hbm.at[wid], ids_v)
        half = b_per_w // 2
        pos_cp = pltpu.async_copy(
            pos_hbm.at[pl.ds(pos_base, half)], pos_v, psem
        )

        def fire16(i, carry):
            vec = ids_v[pl.ds(i * _LANES, _LANES)]
            for l in range(_LANES):
                tok = vec[l]
                pltpu.async_copy(
                    tok_hbm.at[pl.ds(tok, 1)],
                    rows_v.at[pl.ds(i * _LANES + l, 1)],
                    rsems[l % 8],
                )
            return carry

        lax.fori_loop(0, b_per_w // _LANES, fire16, 0)

        for q in range(8):
            pltpu.make_async_copy(
                tok_hbm.at[pl.ds(0, b_per_w // 8)],
                rows_v.at[pl.ds(q * (b_per_w // 8), b_per_w // 8)],
                rsems[q],
            ).wait()

        def half_pass(h):
            hbase = h * half
            pos_cp_h = pltpu.make_async_copy(
                pos_hbm.at[pl.ds(pos_base + hbase, half)], pos_v, psem
            )
            pos_cp_h.wait()
            if h == 0:
                pass

            def add_row(i, carry):
                for c in range(d // _LANES):
                    sl = pl.ds(c * _LANES, _LANES)
                    rows_v[hbase + i, sl] = rows_v[hbase + i, sl] + pos_v[i, sl]
                return carry

            lax.fori_loop(0, half, add_row, 0)

        half_pass(0)
        pltpu.async_copy(pos_hbm.at[pl.ds(pos_base + half, half)], pos_v, psem)
        half_pass(1)

        pltpu.sync_copy(rows_v, out_hbm.at[pl.ds(base, b_per_w)])

    return emb_kernel


def kernel(token_ids, token_table, pos_table):
    batch, seq = token_ids.shape
    d = token_table.shape[1]
    b_per_w = (batch * seq) // _NW
    ids = token_ids.astype(jnp.int32).reshape(_NW, b_per_w)
    out = _build(batch, seq, d)(ids, token_table, pos_table)
    return out.reshape(batch, seq, d)
